# baseline (speedup 1.0000x reference)
import functools

import numpy as np
import jax
import jax.numpy as jnp
from jax import lax
from jax.experimental import pallas as pl
from jax.experimental.pallas import tpu as pltpu
from jax.experimental.pallas import tpu_sc as plsc

_N = 10000
_E = 320000
_D = 128
_NRBF = 16
_CUTOFF = 5.0
_EPS = 1e-15

_NC, _NS, _L = 2, 16, 16
_NW = _NC * _NS

_D2_ROWS = 2560
_EPAD = _D2_ROWS * 128
_D2_RPT = _D2_ROWS // _NW

_EPW = _E // _NW
_CHUNK = 80
_NCHUNK = _EPW // _CHUNK
_ZR = 40
_NZCH = _N // _ZR

_sc_params = pltpu.CompilerParams(needs_layout_passes=False)


@functools.cache
def _sc_mesh():
    return plsc.VectorSubcoreMesh(core_axis_name="c", subcore_axis_name="s")


def _sc_d2_body(x_hbm, y_hbm, z_hbm, src_hbm, dst_hbm, d2_hbm,
                x_v, y_v, z_v, src_v, dst_v, d2_v):
    c = lax.axis_index("c")
    s = lax.axis_index("s")
    w = s * _NC + c
    pltpu.sync_copy(x_hbm, x_v)
    pltpu.sync_copy(y_hbm, y_v)
    pltpu.sync_copy(z_hbm, z_v)
    ebase = w * (_D2_RPT * 128)
    pltpu.sync_copy(src_hbm.at[pl.ds(ebase, _D2_RPT * 128)], src_v)
    pltpu.sync_copy(dst_hbm.at[pl.ds(ebase, _D2_RPT * 128)], dst_v)

    def row(r, carry):
        for q in range(8):
            off = r * 128 + q * 16
            si = src_v[pl.ds(off, 16)]
            di = dst_v[pl.ds(off, 16)]
            dx = plsc.load_gather(x_v, [di]) - plsc.load_gather(x_v, [si])
            dy = plsc.load_gather(y_v, [di]) - plsc.load_gather(y_v, [si])
            dz = plsc.load_gather(z_v, [di]) - plsc.load_gather(z_v, [si])
            d2_v[r, pl.ds(q * 16, 16)] = dx * dx + dy * dy + dz * dz + 3.0 * _EPS
        return carry

    lax.fori_loop(0, _D2_RPT, row, 0)
    pltpu.sync_copy(d2_v, d2_hbm.at[pl.ds(w * _D2_RPT, _D2_RPT)])


@functools.cache
def _sc_d2():
  return pl.kernel(
    _sc_d2_body,
    out_type=jax.ShapeDtypeStruct((_D2_ROWS, 128), jnp.float32),
    mesh=_sc_mesh(),
    scratch_types=[
        pltpu.VMEM((_N,), jnp.float32),
        pltpu.VMEM((_N,), jnp.float32),
        pltpu.VMEM((_N,), jnp.float32),
        pltpu.VMEM((_D2_RPT * 128,), jnp.int32),
        pltpu.VMEM((_D2_RPT * 128,), jnp.int32),
        pltpu.VMEM((_D2_RPT, 128), jnp.float32),
    ],
    compiler_params=_sc_params,
  )


def _sc_edge_body(phi_hbm, ws_hbm, src_hbm, dst_hbm, acc_hbm,
                  acc_sh, rows0, rows1, wsb0, wsb1,
                  is0, is1, is2, is3, id0, id1, id2, id3, zb_v,
                  gs0, gs1, wt0, wt1, ss0, ss1, ix0, ix1, ix2, ix3):
    c = lax.axis_index("c")
    s = lax.axis_index("s")
    w = s * _NC + c
    rows = (rows0, rows1)
    wsb = (wsb0, wsb1)
    isrc = (is0, is1, is2, is3)
    idst = (id0, id1, id2, id3)
    gsem = (gs0, gs1)
    wsem = (wt0, wt1)
    ssem = (ss0, ss1)
    isem = (ix0, ix1, ix2, ix3)
    ebase = w * _EPW

    def _idx_fetch(k, q):
        sl = pl.ds(ebase + k * _CHUNK, _CHUNK)
        pltpu.async_copy(src_hbm.at[sl], isrc[q], isem[q])
        pltpu.async_copy(dst_hbm.at[sl], idst[q], isem[q])

    def _idx_wait(k, q):
        sl = pl.ds(ebase + k * _CHUNK, _CHUNK)
        pltpu.make_async_copy(src_hbm.at[sl], isrc[q], isem[q]).wait()
        pltpu.make_async_copy(dst_hbm.at[sl], idst[q], isem[q]).wait()

    def _start(q, b):
        pltpu.async_copy(phi_hbm.at[idst[q]], rows[b], gsem[b])

    def _wrow(k):
        return pl.multiple_of((ebase + k * _CHUNK) // 2, 8)

    def _ws_start(k, b):
        pltpu.async_copy(ws_hbm.at[pl.ds(_wrow(k), _CHUNK // 2)],
                         wsb[b], wsem[b])

    def _wait_in(k, q, b):
        pltpu.make_async_copy(phi_hbm.at[idst[q]], rows[b], gsem[b]).wait()
        pltpu.make_async_copy(ws_hbm.at[pl.ds(_wrow(k), _CHUNK // 2)],
                              wsb[b], wsem[b]).wait()

    def _scatter_start(q, b):
        pltpu.async_copy(rows[b], acc_sh.at[isrc[q]], ssem[b], add=True)

    def _scatter_wait(b):
        pltpu.make_async_copy(rows[b], acc_sh.at[isrc[0]], ssem[b]).wait()

    _idx_fetch(0, 0)
    _idx_fetch(1, 1)
    _idx_wait(0, 0)
    _start(0, 0)
    _ws_start(0, 0)

    def zrow(r, carry):
        for q in range(8):
            zb_v[r, pl.ds(q * 16, 16)] = jnp.zeros((16,), jnp.float32)
        return carry

    lax.fori_loop(0, _ZR, zrow, 0)
    for m in range(_NZCH // _NS + 1):
        cid = s + _NS * m

        @pl.when(cid < _NZCH)
        def _():
            pltpu.sync_copy(zb_v, acc_sh.at[pl.ds(cid * _ZR, _ZR)])
    plsc.subcore_barrier()

    def _mult(b):
        def mpair(m, carry2):
            for qq in range(8):
                sl = pl.ds(qq * 16, 16)
                wv = plsc.bitcast(wsb[b][m, sl], jnp.bfloat16)
                wa, wo = plsc.unpack(wv, format=plsc.PackFormat.INTERLEAVED)
                rows[b][2 * m, sl] = rows[b][2 * m, sl] * wa
                rows[b][2 * m + 1, sl] = rows[b][2 * m + 1, sl] * wo
            return carry2

        lax.fori_loop(0, _CHUNK // 2, mpair, 0)

    _NSTEADY = (_NCHUNK - 5) // 4

    def quad(j, carry):
        for b in range(4):
            k = 4 * j + b
            rb = b % 2
            nb = 1 - rb
            q1 = (b + 1) % 4
            q2 = (b + 2) % 4

            @pl.when(k >= 1)
            def _():
                _scatter_wait(nb)
            _idx_wait(k + 1, q1)
            _start(q1, nb)
            _ws_start(k + 1, nb)
            _idx_fetch(k + 2, q2)
            _wait_in(k, b, rb)
            _mult(rb)
            _scatter_start(b, rb)
        return carry

    lax.fori_loop(0, _NSTEADY, quad, 0)
    for k in range(4 * _NSTEADY, _NCHUNK):
        rb = k % 2
        nb = 1 - rb
        _scatter_wait(nb)
        if k + 1 < _NCHUNK:
            _idx_wait(k + 1, (k + 1) % 4)
            _start((k + 1) % 4, nb)
            _ws_start(k + 1, nb)
        if k + 2 < _NCHUNK:
            _idx_fetch(k + 2, (k + 2) % 4)
        _wait_in(k, k % 4, rb)
        _mult(rb)
        _scatter_start(k % 4, rb)
    _scatter_wait((_NCHUNK - 1) % 2)
    plsc.subcore_barrier()
    for m in range(_NZCH // _NS + 1):
        cid = s + _NS * m

        @pl.when(cid < _NZCH)
        def _():
            r0 = cid * _ZR
            pltpu.sync_copy(acc_sh.at[pl.ds(r0, _ZR)], zb_v)
            pltpu.sync_copy(zb_v, acc_hbm.at[c, pl.ds(r0, _ZR)])


@functools.cache
def _sc_edge():
  return pl.kernel(
    _sc_edge_body,
    out_type=jax.ShapeDtypeStruct((_NC, _N, _D), jnp.float32),
    mesh=_sc_mesh(),
    scratch_types=(
        [pltpu.VMEM_SHARED((_N, _D), jnp.float32)]
        + [pltpu.VMEM((_CHUNK, _D), jnp.float32)] * 2
        + [pltpu.VMEM((_CHUNK // 2, _D), jnp.uint32)] * 2
        + [pltpu.VMEM((_CHUNK,), jnp.int32)] * 8
        + [pltpu.VMEM((_ZR, _D), jnp.float32)]
        + [pltpu.SemaphoreType.DMA] * 10
    ),
    compiler_params=_sc_params,
  )


_BR = 1000


def _mlp2_block(x, w1t, b1, w2t, b2):
    h = jnp.maximum(x, 0.0)
    h = jnp.dot(h, w1t, preferred_element_type=jnp.float32) + b1
    h = jnp.maximum(h, 0.0)
    return jnp.dot(h, w2t, preferred_element_type=jnp.float32) + b2


def _phi_body(x_ref, w1t, b1, w2t, b2, o_ref):
    x = x_ref[...]
    h = jnp.dot(x, w1t[...], preferred_element_type=jnp.float32) + b1[...]
    h = h * jax.nn.sigmoid(h)
    o_ref[...] = jnp.dot(h, w2t[...], preferred_element_type=jnp.float32) + b2[...]


def _upd_body(s_ref, acc_ref, dw1t, db1, dw2t, db2, mw1t, mb1, mw2t, mb2,
              vi_ref, phi_ref):
    v = acc_ref[0] + acc_ref[1]
    vi = s_ref[...] + _mlp2_block(v, dw1t[...], db1[...], dw2t[...], db2[...])
    vi_ref[...] = vi
    h = jnp.dot(vi, mw1t[...], preferred_element_type=jnp.float32) + mb1[...]
    h = h * jax.nn.sigmoid(h)
    phi_ref[...] = jnp.dot(h, mw2t[...], preferred_element_type=jnp.float32) + mb2[...]


def _fin_body(vi_ref, acc_ref, dw1t, db1, dw2t, db2, fw1t, fb1, fw2t, fb2,
              o_ref):
    v = acc_ref[0] + acc_ref[1]
    vi2 = vi_ref[...] + _mlp2_block(v, dw1t[...], db1[...], dw2t[...], db2[...])
    o_ref[...] = _mlp2_block(vi2, fw1t[...], fb1[...], fw2t[...], fb2[...])


_WSR = 32

_SIN_C = (1.0000000e+00, -1.6666648e-01, 8.3329687e-03, -1.9821891e-04,
          2.7130318e-06, -2.0843258e-08)
_COS_C = (1.0000000e+00, -4.9999997e-01, 4.1666560e-02, -1.3888067e-03,
          2.4774410e-05, -2.7114876e-07, 1.7351648e-09)


def _pack_pairs(w):
    lo = lax.bitcast_convert_type(w[:64, :].astype(jnp.bfloat16),
                                  jnp.uint16).astype(jnp.uint32)
    hi = lax.bitcast_convert_type(w[64:, :].astype(jnp.bfloat16),
                                  jnp.uint16).astype(jnp.uint32)
    return lo | (hi << 16)


def _ws_body(d2_ref, dw1t_1, db1_1, dw1t_2, db1_2, ws1_ref, ws2_ref):
    f32 = jnp.float32
    ones = jnp.ones((1, 1), f32)
    d2b = d2_ref[...]
    dist = jnp.sqrt(d2b)
    theta = np.float32(np.pi / _CUTOFF) * jnp.minimum(dist, _CUTOFF)
    t2 = theta * theta
    ps = jnp.full_like(t2, _SIN_C[-1])
    for c in _SIN_C[-2::-1]:
        ps = ps * t2 + c
    s1 = theta * ps
    pc = jnp.full_like(t2, _COS_C[-1])
    for c in _COS_C[-2::-1]:
        pc = pc * t2 + c
    c1 = pc
    env = jnp.where(dist <= _CUTOFF, 0.5 * (c1 + 1.0), 0.0)
    inv_d = 1.0 / dist
    sins = [s1, 2.0 * c1 * s1]
    for _ in range(3, _NRBF + 1):
        sins.append(2.0 * c1 * sins[-1] - sins[-2])
    cn = (((0,), (0,)), ((), ()))
    for r in range(_WSR):
        S = jnp.concatenate([sins[k][r:r + 1, :] for k in range(_NRBF)],
                            axis=0)
        S = S * inv_d[r:r + 1, :]
        envc = lax.dot_general(env[r:r + 1, :], ones, cn,
                               preferred_element_type=f32)
        w1 = lax.dot_general(S, dw1t_1[...], cn,
                             preferred_element_type=f32) + db1_1[...]
        w2 = lax.dot_general(S, dw1t_2[...], cn,
                             preferred_element_type=f32) + db1_2[...]
        ws1_ref[r * 64:(r + 1) * 64, :] = _pack_pairs(w1 * envc)
        ws2_ref[r * 64:(r + 1) * 64, :] = _pack_pairs(w2 * envc)


def _full(shape):
    return pl.BlockSpec(shape, lambda i: tuple(0 for _ in shape))


def _phi_call(x, w1t, b1, w2t, b2):
    return pl.pallas_call(
        _phi_body,
        grid=(_N // _BR,),
        in_specs=[
            pl.BlockSpec((_BR, _D), lambda i: (i, 0)),
            _full((_D, _D)), _full((1, _D)), _full((_D, _D)), _full((1, _D)),
        ],
        out_specs=pl.BlockSpec((_BR, _D), lambda i: (i, 0)),
        out_shape=jax.ShapeDtypeStruct((_N, _D), jnp.float32),
    )(x, w1t, b1, w2t, b2)


def _upd_call(s, acc, dw1t, db1, dw2t, db2, mw1t, mb1, mw2t, mb2):
    return pl.pallas_call(
        _upd_body,
        grid=(_N // _BR,),
        in_specs=[
            pl.BlockSpec((_BR, _D), lambda i: (i, 0)),
            pl.BlockSpec((_NC, _BR, _D), lambda i: (0, i, 0)),
            _full((_D, _D)), _full((1, _D)), _full((_D, _D)), _full((1, _D)),
            _full((_D, _D)), _full((1, _D)), _full((_D, _D)), _full((1, _D)),
        ],
        out_specs=[
            pl.BlockSpec((_BR, _D), lambda i: (i, 0)),
            pl.BlockSpec((_BR, _D), lambda i: (i, 0)),
        ],
        out_shape=[
            jax.ShapeDtypeStruct((_N, _D), jnp.float32),
            jax.ShapeDtypeStruct((_N, _D), jnp.float32),
        ],
    )(s, acc, dw1t, db1, dw2t, db2, mw1t, mb1, mw2t, mb2)


def _fin_call(vi, acc, dw1t, db1, dw2t, db2, fw1t, fb1, fw2t, fb2):
    return pl.pallas_call(
        _fin_body,
        grid=(_N // _BR,),
        in_specs=[
            pl.BlockSpec((_BR, _D), lambda i: (i, 0)),
            pl.BlockSpec((_NC, _BR, _D), lambda i: (0, i, 0)),
            _full((_D, _D)), _full((1, _D)), _full((_D, _D)), _full((1, _D)),
            _full((_D, _D)), _full((1, _D)), _full((_D, _D)), _full((1, _D)),
        ],
        out_specs=pl.BlockSpec((_BR, _D), lambda i: (i, 0)),
        out_shape=jax.ShapeDtypeStruct((_N, _D), jnp.float32),
    )(vi, acc, dw1t, db1, dw2t, db2, fw1t, fb1, fw2t, fb2)


def _ws_call(d2, dw1t_1, db1_1, dw1t_2, db1_2):
    return pl.pallas_call(
        _ws_body,
        grid=(_D2_ROWS // _WSR,),
        in_specs=[
            pl.BlockSpec((_WSR, 128), lambda i: (i, 0)),
            _full((_NRBF, _D)), _full((1, _D)),
            _full((_NRBF, _D)), _full((1, _D)),
        ],
        out_specs=[
            pl.BlockSpec((_WSR * 64, _D), lambda i: (i, 0)),
            pl.BlockSpec((_WSR * 64, _D), lambda i: (i, 0)),
        ],
        out_shape=[
            jax.ShapeDtypeStruct((_EPAD // 2, _D), jnp.uint32),
            jax.ShapeDtypeStruct((_EPAD // 2, _D), jnp.uint32),
        ],
    )(d2, dw1t_1, db1_1, dw1t_2, db1_2)


def kernel(cg_xyz, CG_nbr_list, mapping, S,
           im1_w1, im1_b1, im1_w2, im1_b2, im1_dw, im1_db,
           im2_w1, im2_b1, im2_w2, im2_b2, im2_dw, im2_db,
           d1_w1, d1_b1, d1_w2, d1_b2,
           d2_w1, d2_b1, d2_w2, d2_b2,
           d3_w1, d3_b1, d3_w2, d3_b2):
    f32 = jnp.float32
    x = cg_xyz[:, 0].astype(f32)
    y = cg_xyz[:, 1].astype(f32)
    z = cg_xyz[:, 2].astype(f32)
    src = CG_nbr_list[:, 0].astype(jnp.int32)
    dst = CG_nbr_list[:, 1].astype(jnp.int32)
    srcp = jnp.pad(src, (0, _EPAD - _E))
    dstp = jnp.pad(dst, (0, _EPAD - _E))
    lin = jnp.arange(_E, dtype=jnp.int32)
    qq = lin % 128
    perm = (lin - qq) + (qq // 2) + 64 * (qq % 2)
    srcq = src[perm]
    dstq = dst[perm]

    d2 = _sc_d2()(x, y, z, srcp, dstp)

    row = lambda b: b.reshape(1, -1).astype(f32)
    ws1, ws2 = _ws_call(d2, im1_dw.T.astype(f32), row(im1_db),
                        im2_dw.T.astype(f32), row(im2_db))

    phi1 = _phi_call(S, im1_w1.T, row(im1_b1), im1_w2.T, row(im1_b2))
    acc1 = _sc_edge()(phi1, ws1, srcq, dstq)

    vi, phi2 = _upd_call(S, acc1,
                         d1_w1.T, row(d1_b1), d1_w2.T, row(d1_b2),
                         im2_w1.T, row(im2_b1), im2_w2.T, row(im2_b2))
    acc2 = _sc_edge()(phi2, ws2, srcq, dstq)

    fw1t = jnp.zeros((_D, _D), f32).at[:, :39].set(d3_w1.T)
    fb1 = jnp.zeros((1, _D), f32).at[0, :39].set(d3_b1)
    fw2t = jnp.zeros((_D, _D), f32).at[:39, :39].set(d3_w2.T)
    fb2 = jnp.zeros((1, _D), f32).at[0, :39].set(d3_b2)
    vfull = _fin_call(vi, acc2,
                      d2_w1.T, row(d2_b1), d2_w2.T, row(d2_b2),
                      fw1t, fb1, fw2t, fb2)
    V = vfull[:, :39].reshape(_N, 13, 3)
    return (None, V)


# speedup vs baseline: 1.3450x; 1.3450x over previous
import functools

import numpy as np
import jax
import jax.numpy as jnp
from jax import lax
from jax.experimental import pallas as pl
from jax.experimental.pallas import tpu as pltpu
from jax.experimental.pallas import tpu_sc as plsc

_N = 10000
_E = 320000
_D = 128
_NRBF = 16
_CUTOFF = 5.0
_EPS = 1e-15

_NC, _NS, _L = 2, 16, 16
_NW = _NC * _NS

_D2_ROWS = 2560
_EPAD = _D2_ROWS * 128
_D2_RPT = _D2_ROWS // _NW

_EPW = _E // _NW
_CHUNK = 80
_NCHUNK = _EPW // _CHUNK
_ZR = 40
_NZCH = _N // _ZR

_sc_params = pltpu.CompilerParams(needs_layout_passes=False)


@functools.cache
def _sc_mesh():
    return plsc.VectorSubcoreMesh(core_axis_name="c", subcore_axis_name="s")


def _sc_d2_body(x_hbm, y_hbm, z_hbm, src_hbm, dst_hbm, d2_hbm,
                x_v, y_v, z_v, src_v, dst_v, d2_v):
    c = lax.axis_index("c")
    s = lax.axis_index("s")
    w = s * _NC + c
    pltpu.sync_copy(x_hbm, x_v)
    pltpu.sync_copy(y_hbm, y_v)
    pltpu.sync_copy(z_hbm, z_v)
    ebase = w * (_D2_RPT * 128)
    pltpu.sync_copy(src_hbm.at[pl.ds(ebase, _D2_RPT * 128)], src_v)
    pltpu.sync_copy(dst_hbm.at[pl.ds(ebase, _D2_RPT * 128)], dst_v)

    def row(r, carry):
        for q in range(8):
            off = r * 128 + q * 16
            si = src_v[pl.ds(off, 16)]
            di = dst_v[pl.ds(off, 16)]
            dx = plsc.load_gather(x_v, [di]) - plsc.load_gather(x_v, [si])
            dy = plsc.load_gather(y_v, [di]) - plsc.load_gather(y_v, [si])
            dz = plsc.load_gather(z_v, [di]) - plsc.load_gather(z_v, [si])
            d2_v[r, pl.ds(q * 16, 16)] = dx * dx + dy * dy + dz * dz + 3.0 * _EPS
        return carry

    lax.fori_loop(0, _D2_RPT, row, 0)
    pltpu.sync_copy(d2_v, d2_hbm.at[pl.ds(w * _D2_RPT, _D2_RPT)])


@functools.cache
def _sc_d2():
  return pl.kernel(
    _sc_d2_body,
    out_type=jax.ShapeDtypeStruct((_D2_ROWS, 128), jnp.float32),
    mesh=_sc_mesh(),
    scratch_types=[
        pltpu.VMEM((_N,), jnp.float32),
        pltpu.VMEM((_N,), jnp.float32),
        pltpu.VMEM((_N,), jnp.float32),
        pltpu.VMEM((_D2_RPT * 128,), jnp.int32),
        pltpu.VMEM((_D2_RPT * 128,), jnp.int32),
        pltpu.VMEM((_D2_RPT, 128), jnp.float32),
    ],
    compiler_params=_sc_params,
  )


def _sc_edge_body(phi_hbm, ws_hbm, src_hbm, dst_hbm, acc_hbm,
                  acc_sh, rows0, rows1, wsb0, wsb1,
                  is0, is1, is2, is3, id0, id1, id2, id3, zb_v,
                  gs0, gs1, wt0, wt1, ss0, ss1, ix0, ix1, ix2, ix3):
    c = lax.axis_index("c")
    s = lax.axis_index("s")
    w = s * _NC + c
    rows = (rows0, rows1)
    wsb = (wsb0, wsb1)
    isrc = (is0, is1, is2, is3)
    idst = (id0, id1, id2, id3)
    gsem = (gs0, gs1)
    wsem = (wt0, wt1)
    ssem = (ss0, ss1)
    isem = (ix0, ix1, ix2, ix3)
    ebase = w * _EPW

    def _idx_fetch(k, q):
        sl = pl.ds(ebase + k * _CHUNK, _CHUNK)
        pltpu.async_copy(src_hbm.at[sl], isrc[q], isem[q])
        pltpu.async_copy(dst_hbm.at[sl], idst[q], isem[q])

    def _idx_wait(k, q):
        sl = pl.ds(ebase + k * _CHUNK, _CHUNK)
        pltpu.make_async_copy(src_hbm.at[sl], isrc[q], isem[q]).wait()
        pltpu.make_async_copy(dst_hbm.at[sl], idst[q], isem[q]).wait()

    def _start(q, b):
        pltpu.async_copy(phi_hbm.at[idst[q]], rows[b], gsem[b])

    def _ws_start(k, b):
        pltpu.async_copy(ws_hbm.at[pl.ds(ebase + k * _CHUNK, _CHUNK)],
                         wsb[b], wsem[b])

    def _wait_in(k, q, b):
        pltpu.make_async_copy(phi_hbm.at[idst[q]], rows[b], gsem[b]).wait()
        pltpu.make_async_copy(ws_hbm.at[pl.ds(ebase + k * _CHUNK, _CHUNK)],
                              wsb[b], wsem[b]).wait()

    def _scatter_start(q, b):
        pltpu.async_copy(rows[b], acc_sh.at[isrc[q]], ssem[b], add=True)

    def _scatter_wait(b):
        pltpu.make_async_copy(rows[b], acc_sh.at[isrc[0]], ssem[b]).wait()

    _idx_fetch(0, 0)
    _idx_fetch(1, 1)
    _idx_wait(0, 0)
    _start(0, 0)
    _ws_start(0, 0)

    def zrow(r, carry):
        for q in range(8):
            zb_v[r, pl.ds(q * 16, 16)] = jnp.zeros((16,), jnp.float32)
        return carry

    lax.fori_loop(0, _ZR, zrow, 0)
    for m in range(_NZCH // _NS + 1):
        cid = s + _NS * m

        @pl.when(cid < _NZCH)
        def _():
            pltpu.sync_copy(zb_v, acc_sh.at[pl.ds(cid * _ZR, _ZR)])
    plsc.subcore_barrier()

    def _mult(b):
        def mrow(r, carry2):
            for qq in range(8):
                sl = pl.ds(qq * 16, 16)
                rows[b][r, sl] = rows[b][r, sl] * wsb[b][r, sl]
            return carry2

        lax.fori_loop(0, _CHUNK, mrow, 0)

    _NSTEADY = (_NCHUNK - 5) // 4

    def quad(j, carry):
        for b in range(4):
            k = 4 * j + b
            rb = b % 2
            nb = 1 - rb
            q1 = (b + 1) % 4
            q2 = (b + 2) % 4

            @pl.when(k >= 1)
            def _():
                _scatter_wait(nb)
            _idx_wait(k + 1, q1)
            _start(q1, nb)
            _ws_start(k + 1, nb)
            _idx_fetch(k + 2, q2)
            _wait_in(k, b, rb)
            _mult(rb)
            _scatter_start(b, rb)
        return carry

    lax.fori_loop(0, _NSTEADY, quad, 0)
    for k in range(4 * _NSTEADY, _NCHUNK):
        rb = k % 2
        nb = 1 - rb
        _scatter_wait(nb)
        if k + 1 < _NCHUNK:
            _idx_wait(k + 1, (k + 1) % 4)
            _start((k + 1) % 4, nb)
            _ws_start(k + 1, nb)
        if k + 2 < _NCHUNK:
            _idx_fetch(k + 2, (k + 2) % 4)
        _wait_in(k, k % 4, rb)
        _mult(rb)
        _scatter_start(k % 4, rb)
    _scatter_wait((_NCHUNK - 1) % 2)
    plsc.subcore_barrier()
    for m in range(_NZCH // _NS + 1):
        cid = s + _NS * m

        @pl.when(cid < _NZCH)
        def _():
            r0 = cid * _ZR
            pltpu.sync_copy(acc_sh.at[pl.ds(r0, _ZR)], zb_v)
            pltpu.sync_copy(zb_v, acc_hbm.at[c, pl.ds(r0, _ZR)])


@functools.cache
def _sc_edge():
  return pl.kernel(
    _sc_edge_body,
    out_type=jax.ShapeDtypeStruct((_NC, _N, _D), jnp.float32),
    mesh=_sc_mesh(),
    scratch_types=(
        [pltpu.VMEM_SHARED((_N, _D), jnp.float32)]
        + [pltpu.VMEM((_CHUNK, _D), jnp.float32)] * 4
        + [pltpu.VMEM((_CHUNK,), jnp.int32)] * 8
        + [pltpu.VMEM((_ZR, _D), jnp.float32)]
        + [pltpu.SemaphoreType.DMA] * 10
    ),
    compiler_params=_sc_params,
  )


_BR = 1000


def _mlp2_block(x, w1t, b1, w2t, b2):
    h = jnp.maximum(x, 0.0)
    h = jnp.dot(h, w1t, preferred_element_type=jnp.float32) + b1
    h = jnp.maximum(h, 0.0)
    return jnp.dot(h, w2t, preferred_element_type=jnp.float32) + b2


def _phi_body(x_ref, w1t, b1, w2t, b2, o_ref):
    x = x_ref[...]
    h = jnp.dot(x, w1t[...], preferred_element_type=jnp.float32) + b1[...]
    h = h * jax.nn.sigmoid(h)
    o_ref[...] = jnp.dot(h, w2t[...], preferred_element_type=jnp.float32) + b2[...]


def _upd_body(s_ref, acc_ref, dw1t, db1, dw2t, db2, mw1t, mb1, mw2t, mb2,
              vi_ref, phi_ref):
    v = acc_ref[0] + acc_ref[1]
    vi = s_ref[...] + _mlp2_block(v, dw1t[...], db1[...], dw2t[...], db2[...])
    vi_ref[...] = vi
    h = jnp.dot(vi, mw1t[...], preferred_element_type=jnp.float32) + mb1[...]
    h = h * jax.nn.sigmoid(h)
    phi_ref[...] = jnp.dot(h, mw2t[...], preferred_element_type=jnp.float32) + mb2[...]


def _fin_body(vi_ref, acc_ref, dw1t, db1, dw2t, db2, fw1t, fb1, fw2t, fb2,
              o_ref):
    v = acc_ref[0] + acc_ref[1]
    vi2 = vi_ref[...] + _mlp2_block(v, dw1t[...], db1[...], dw2t[...], db2[...])
    o_ref[...] = _mlp2_block(vi2, fw1t[...], fb1[...], fw2t[...], fb2[...])


_WSR = 32

_SIN_C = (1.0000000e+00, -1.6666648e-01, 8.3329687e-03, -1.9821891e-04,
          2.7130318e-06, -2.0843258e-08)
_COS_C = (1.0000000e+00, -4.9999997e-01, 4.1666560e-02, -1.3888067e-03,
          2.4774410e-05, -2.7114876e-07, 1.7351648e-09)


def _rbf_body(d2_ref, planes_ref, env_ref):
    d2b = d2_ref[...]
    dist = jnp.sqrt(d2b)
    theta = np.float32(np.pi / _CUTOFF) * jnp.minimum(dist, _CUTOFF)
    t2 = theta * theta
    ps = jnp.full_like(t2, _SIN_C[-1])
    for c in _SIN_C[-2::-1]:
        ps = ps * t2 + c
    s1 = theta * ps
    pc = jnp.full_like(t2, _COS_C[-1])
    for c in _COS_C[-2::-1]:
        pc = pc * t2 + c
    c1 = pc
    env_ref[...] = jnp.where(dist <= _CUTOFF, 0.5 * (c1 + 1.0), 0.0)
    inv_d = 1.0 / dist
    sins = [s1, 2.0 * c1 * s1]
    for _ in range(3, _NRBF + 1):
        sins.append(2.0 * c1 * sins[-1] - sins[-2])
    for k in range(_NRBF):
        planes_ref[k] = sins[k] * inv_d


def _wsk_body(planes_ref, env_ref, dw1t, db1, ws_ref):
    f32 = jnp.float32
    ones = jnp.ones((1, 1), f32)
    cn = (((0,), (0,)), ((), ()))
    for r in range(_WSR):
        S = planes_ref[:, r, :]
        envc = lax.dot_general(env_ref[r:r + 1, :], ones, cn,
                               preferred_element_type=f32)
        w = lax.dot_general(S, dw1t[...], cn,
                            preferred_element_type=f32) + db1[...]
        ws_ref[r * 128:(r + 1) * 128, :] = w * envc


def _full(shape):
    return pl.BlockSpec(shape, lambda i: tuple(0 for _ in shape))


def _phi_call(x, w1t, b1, w2t, b2):
    return pl.pallas_call(
        _phi_body,
        grid=(_N // _BR,),
        in_specs=[
            pl.BlockSpec((_BR, _D), lambda i: (i, 0)),
            _full((_D, _D)), _full((1, _D)), _full((_D, _D)), _full((1, _D)),
        ],
        out_specs=pl.BlockSpec((_BR, _D), lambda i: (i, 0)),
        out_shape=jax.ShapeDtypeStruct((_N, _D), jnp.float32),
    )(x, w1t, b1, w2t, b2)


def _upd_call(s, acc, dw1t, db1, dw2t, db2, mw1t, mb1, mw2t, mb2):
    return pl.pallas_call(
        _upd_body,
        grid=(_N // _BR,),
        in_specs=[
            pl.BlockSpec((_BR, _D), lambda i: (i, 0)),
            pl.BlockSpec((_NC, _BR, _D), lambda i: (0, i, 0)),
            _full((_D, _D)), _full((1, _D)), _full((_D, _D)), _full((1, _D)),
            _full((_D, _D)), _full((1, _D)), _full((_D, _D)), _full((1, _D)),
        ],
        out_specs=[
            pl.BlockSpec((_BR, _D), lambda i: (i, 0)),
            pl.BlockSpec((_BR, _D), lambda i: (i, 0)),
        ],
        out_shape=[
            jax.ShapeDtypeStruct((_N, _D), jnp.float32),
            jax.ShapeDtypeStruct((_N, _D), jnp.float32),
        ],
    )(s, acc, dw1t, db1, dw2t, db2, mw1t, mb1, mw2t, mb2)


def _fin_call(vi, acc, dw1t, db1, dw2t, db2, fw1t, fb1, fw2t, fb2):
    return pl.pallas_call(
        _fin_body,
        grid=(_N // _BR,),
        in_specs=[
            pl.BlockSpec((_BR, _D), lambda i: (i, 0)),
            pl.BlockSpec((_NC, _BR, _D), lambda i: (0, i, 0)),
            _full((_D, _D)), _full((1, _D)), _full((_D, _D)), _full((1, _D)),
            _full((_D, _D)), _full((1, _D)), _full((_D, _D)), _full((1, _D)),
        ],
        out_specs=pl.BlockSpec((_BR, _D), lambda i: (i, 0)),
        out_shape=jax.ShapeDtypeStruct((_N, _D), jnp.float32),
    )(vi, acc, dw1t, db1, dw2t, db2, fw1t, fb1, fw2t, fb2)


def _rbf_call(d2):
    return pl.pallas_call(
        _rbf_body,
        grid=(_D2_ROWS // _WSR,),
        in_specs=[pl.BlockSpec((_WSR, 128), lambda i: (i, 0))],
        out_specs=[
            pl.BlockSpec((_NRBF, _WSR, 128), lambda i: (0, i, 0)),
            pl.BlockSpec((_WSR, 128), lambda i: (i, 0)),
        ],
        out_shape=[
            jax.ShapeDtypeStruct((_NRBF, _D2_ROWS, 128), jnp.float32),
            jax.ShapeDtypeStruct((_D2_ROWS, 128), jnp.float32),
        ],
    )(d2)


def _wsk_call(planes, env, dw1t, db1):
    return pl.pallas_call(
        _wsk_body,
        grid=(_D2_ROWS // _WSR,),
        in_specs=[
            pl.BlockSpec((_NRBF, _WSR, 128), lambda i: (0, i, 0)),
            pl.BlockSpec((_WSR, 128), lambda i: (i, 0)),
            _full((_NRBF, _D)), _full((1, _D)),
        ],
        out_specs=pl.BlockSpec((_WSR * 128, _D), lambda i: (i, 0)),
        out_shape=jax.ShapeDtypeStruct((_EPAD, _D), jnp.float32),
    )(planes, env, dw1t, db1)


def kernel(cg_xyz, CG_nbr_list, mapping, S,
           im1_w1, im1_b1, im1_w2, im1_b2, im1_dw, im1_db,
           im2_w1, im2_b1, im2_w2, im2_b2, im2_dw, im2_db,
           d1_w1, d1_b1, d1_w2, d1_b2,
           d2_w1, d2_b1, d2_w2, d2_b2,
           d3_w1, d3_b1, d3_w2, d3_b2):
    f32 = jnp.float32
    x = cg_xyz[:, 0].astype(f32)
    y = cg_xyz[:, 1].astype(f32)
    z = cg_xyz[:, 2].astype(f32)
    src = CG_nbr_list[:, 0].astype(jnp.int32)
    dst = CG_nbr_list[:, 1].astype(jnp.int32)
    srcp = jnp.pad(src, (0, _EPAD - _E))
    dstp = jnp.pad(dst, (0, _EPAD - _E))

    d2 = _sc_d2()(x, y, z, srcp, dstp)

    row = lambda b: b.reshape(1, -1).astype(f32)
    planes, env = _rbf_call(d2)
    ws1 = _wsk_call(planes, env, im1_dw.T.astype(f32), row(im1_db))

    phi1 = _phi_call(S, im1_w1.T, row(im1_b1), im1_w2.T, row(im1_b2))
    acc1 = _sc_edge()(phi1, ws1, src, dst)
    ws2 = _wsk_call(planes, env, im2_dw.T.astype(f32), row(im2_db))

    vi, phi2 = _upd_call(S, acc1,
                         d1_w1.T, row(d1_b1), d1_w2.T, row(d1_b2),
                         im2_w1.T, row(im2_b1), im2_w2.T, row(im2_b2))
    acc2 = _sc_edge()(phi2, ws2, src, dst)

    fw1t = jnp.zeros((_D, _D), f32).at[:, :39].set(d3_w1.T)
    fb1 = jnp.zeros((1, _D), f32).at[0, :39].set(d3_b1)
    fw2t = jnp.zeros((_D, _D), f32).at[:39, :39].set(d3_w2.T)
    fb2 = jnp.zeros((1, _D), f32).at[0, :39].set(d3_b2)
    vfull = _fin_call(vi, acc2,
                      d2_w1.T, row(d2_b1), d2_w2.T, row(d2_b2),
                      fw1t, fb1, fw2t, fb2)
    V = vfull[:, :39].reshape(_N, 13, 3)
    return (None, V)


# speedup vs baseline: 1.4251x; 1.0595x over previous
import functools

import numpy as np
import jax
import jax.numpy as jnp
from jax import lax
from jax.experimental import pallas as pl
from jax.experimental.pallas import tpu as pltpu
from jax.experimental.pallas import tpu_sc as plsc

_N = 10000
_E = 320000
_D = 128
_NRBF = 16
_CUTOFF = 5.0
_EPS = 1e-15

_NC, _NS, _L = 2, 16, 16
_NW = _NC * _NS

_D2_ROWS = 2560
_EPAD = _D2_ROWS * 128
_D2_RPT = _D2_ROWS // _NW

_EPW = _E // _NW
_CHUNK = 80
_NCHUNK = _EPW // _CHUNK
_ZR = 40
_NZCH = _N // _ZR

_sc_params = pltpu.CompilerParams(needs_layout_passes=False)


@functools.cache
def _sc_mesh():
    return plsc.VectorSubcoreMesh(core_axis_name="c", subcore_axis_name="s")


def _sc_d2_body(x_hbm, y_hbm, z_hbm, src_hbm, dst_hbm, d2_hbm,
                x_v, y_v, z_v, src_v, dst_v, d2_v):
    c = lax.axis_index("c")
    s = lax.axis_index("s")
    w = s * _NC + c
    pltpu.sync_copy(x_hbm, x_v)
    pltpu.sync_copy(y_hbm, y_v)
    pltpu.sync_copy(z_hbm, z_v)
    ebase = w * (_D2_RPT * 128)
    pltpu.sync_copy(src_hbm.at[pl.ds(ebase, _D2_RPT * 128)], src_v)
    pltpu.sync_copy(dst_hbm.at[pl.ds(ebase, _D2_RPT * 128)], dst_v)

    def row(r, carry):
        for q in range(8):
            off = r * 128 + q * 16
            si = src_v[pl.ds(off, 16)]
            di = dst_v[pl.ds(off, 16)]
            dx = plsc.load_gather(x_v, [di]) - plsc.load_gather(x_v, [si])
            dy = plsc.load_gather(y_v, [di]) - plsc.load_gather(y_v, [si])
            dz = plsc.load_gather(z_v, [di]) - plsc.load_gather(z_v, [si])
            d2_v[r, pl.ds(q * 16, 16)] = dx * dx + dy * dy + dz * dz + 3.0 * _EPS
        return carry

    lax.fori_loop(0, _D2_RPT, row, 0)
    pltpu.sync_copy(d2_v, d2_hbm.at[pl.ds(w * _D2_RPT, _D2_RPT)])


@functools.cache
def _sc_d2():
  return pl.kernel(
    _sc_d2_body,
    out_type=jax.ShapeDtypeStruct((_D2_ROWS, 128), jnp.float32),
    mesh=_sc_mesh(),
    scratch_types=[
        pltpu.VMEM((_N,), jnp.float32),
        pltpu.VMEM((_N,), jnp.float32),
        pltpu.VMEM((_N,), jnp.float32),
        pltpu.VMEM((_D2_RPT * 128,), jnp.int32),
        pltpu.VMEM((_D2_RPT * 128,), jnp.int32),
        pltpu.VMEM((_D2_RPT, 128), jnp.float32),
    ],
    compiler_params=_sc_params,
  )


def _sc_edge_body(phi_hbm, ws_hbm, src_hbm, dst_hbm, acc_hbm,
                  acc_sh, rows0, rows1, wsb0, wsb1,
                  is0, is1, is2, is3, id0, id1, id2, id3, zb_v,
                  gs0, gs1, wt0, wt1, ss0, ss1, ix0, ix1, ix2, ix3):
    c = lax.axis_index("c")
    s = lax.axis_index("s")
    w = s * _NC + c
    rows = (rows0, rows1)
    wsb = (wsb0, wsb1)
    isrc = (is0, is1, is2, is3)
    idst = (id0, id1, id2, id3)
    gsem = (gs0, gs1)
    wsem = (wt0, wt1)
    ssem = (ss0, ss1)
    isem = (ix0, ix1, ix2, ix3)
    ebase = w * _EPW

    def _idx_fetch(k, q):
        sl = pl.ds(ebase + k * _CHUNK, _CHUNK)
        pltpu.async_copy(src_hbm.at[sl], isrc[q], isem[q])
        pltpu.async_copy(dst_hbm.at[sl], idst[q], isem[q])

    def _idx_wait(k, q):
        sl = pl.ds(ebase + k * _CHUNK, _CHUNK)
        pltpu.make_async_copy(src_hbm.at[sl], isrc[q], isem[q]).wait()
        pltpu.make_async_copy(dst_hbm.at[sl], idst[q], isem[q]).wait()

    def _start(q, b):
        pltpu.async_copy(phi_hbm.at[idst[q]], rows[b], gsem[b])

    def _ws_start(k, b):
        pltpu.async_copy(ws_hbm.at[pl.ds(ebase + k * _CHUNK, _CHUNK)],
                         wsb[b], wsem[b])

    def _wait_in(k, q, b):
        pltpu.make_async_copy(phi_hbm.at[idst[q]], rows[b], gsem[b]).wait()
        pltpu.make_async_copy(ws_hbm.at[pl.ds(ebase + k * _CHUNK, _CHUNK)],
                              wsb[b], wsem[b]).wait()

    def _scatter_start(q, b):
        pltpu.async_copy(rows[b], acc_sh.at[isrc[q]], ssem[b], add=True)

    def _scatter_wait(b):
        pltpu.make_async_copy(rows[b], acc_sh.at[isrc[0]], ssem[b]).wait()

    _idx_fetch(0, 0)
    _idx_fetch(1, 1)
    _idx_wait(0, 0)
    _start(0, 0)
    _ws_start(0, 0)

    def zrow(r, carry):
        for q in range(8):
            zb_v[r, pl.ds(q * 16, 16)] = jnp.zeros((16,), jnp.float32)
        return carry

    lax.fori_loop(0, _ZR, zrow, 0)
    for m in range(_NZCH // _NS + 1):
        cid = s + _NS * m

        @pl.when(cid < _NZCH)
        def _():
            pltpu.sync_copy(zb_v, acc_sh.at[pl.ds(cid * _ZR, _ZR)])
    plsc.subcore_barrier()

    def _mult(b):
        def mrow(r, carry2):
            for qq in range(8):
                sl = pl.ds(qq * 16, 16)
                rows[b][r, sl] = rows[b][r, sl] * wsb[b][r, sl]
            return carry2

        lax.fori_loop(0, _CHUNK, mrow, 0)

    _NSTEADY = (_NCHUNK - 5) // 4

    def quad(j, carry):
        for b in range(4):
            k = 4 * j + b
            rb = b % 2
            nb = 1 - rb
            q1 = (b + 1) % 4
            q2 = (b + 2) % 4

            @pl.when(k >= 1)
            def _():
                _scatter_wait(nb)
            _idx_wait(k + 1, q1)
            _start(q1, nb)
            _ws_start(k + 1, nb)
            _idx_fetch(k + 2, q2)
            _wait_in(k, b, rb)
            _mult(rb)
            _scatter_start(b, rb)
        return carry

    lax.fori_loop(0, _NSTEADY, quad, 0)
    for k in range(4 * _NSTEADY, _NCHUNK):
        rb = k % 2
        nb = 1 - rb
        _scatter_wait(nb)
        if k + 1 < _NCHUNK:
            _idx_wait(k + 1, (k + 1) % 4)
            _start((k + 1) % 4, nb)
            _ws_start(k + 1, nb)
        if k + 2 < _NCHUNK:
            _idx_fetch(k + 2, (k + 2) % 4)
        _wait_in(k, k % 4, rb)
        _mult(rb)
        _scatter_start(k % 4, rb)
    _scatter_wait((_NCHUNK - 1) % 2)
    plsc.subcore_barrier()
    for m in range(_NZCH // _NS + 1):
        cid = s + _NS * m

        @pl.when(cid < _NZCH)
        def _():
            r0 = cid * _ZR
            pltpu.sync_copy(acc_sh.at[pl.ds(r0, _ZR)], zb_v)
            pltpu.sync_copy(zb_v, acc_hbm.at[c, pl.ds(r0, _ZR)])


@functools.cache
def _sc_edge():
  return pl.kernel(
    _sc_edge_body,
    out_type=jax.ShapeDtypeStruct((_NC, _N, _D), jnp.float32),
    mesh=_sc_mesh(),
    scratch_types=(
        [pltpu.VMEM_SHARED((_N, _D), jnp.float32)]
        + [pltpu.VMEM((_CHUNK, _D), jnp.float32)] * 4
        + [pltpu.VMEM((_CHUNK,), jnp.int32)] * 8
        + [pltpu.VMEM((_ZR, _D), jnp.float32)]
        + [pltpu.SemaphoreType.DMA] * 10
    ),
    compiler_params=_sc_params,
  )


_BR = 1000


def _mlp2_block(x, w1t, b1, w2t, b2):
    h = jnp.maximum(x, 0.0)
    h = jnp.dot(h, w1t, preferred_element_type=jnp.float32) + b1
    h = jnp.maximum(h, 0.0)
    return jnp.dot(h, w2t, preferred_element_type=jnp.float32) + b2


def _phi_body(x_ref, w1t, b1, w2t, b2, o_ref):
    x = x_ref[...]
    h = jnp.dot(x, w1t[...], preferred_element_type=jnp.float32) + b1[...]
    h = h * jax.nn.sigmoid(h)
    o_ref[...] = jnp.dot(h, w2t[...], preferred_element_type=jnp.float32) + b2[...]


def _upd_body(s_ref, acc_ref, dw1t, db1, dw2t, db2, mw1t, mb1, mw2t, mb2,
              vi_ref, phi_ref):
    v = acc_ref[0] + acc_ref[1]
    vi = s_ref[...] + _mlp2_block(v, dw1t[...], db1[...], dw2t[...], db2[...])
    vi_ref[...] = vi
    h = jnp.dot(vi, mw1t[...], preferred_element_type=jnp.float32) + mb1[...]
    h = h * jax.nn.sigmoid(h)
    phi_ref[...] = jnp.dot(h, mw2t[...], preferred_element_type=jnp.float32) + mb2[...]


def _fin_body(vi_ref, acc_ref, dw1t, db1, dw2t, db2, fw1t, fb1, fw2t, fb2,
              o_ref):
    v = acc_ref[0] + acc_ref[1]
    vi2 = vi_ref[...] + _mlp2_block(v, dw1t[...], db1[...], dw2t[...], db2[...])
    o_ref[...] = _mlp2_block(vi2, fw1t[...], fb1[...], fw2t[...], fb2[...])


_WSR = 32

_SIN_C = (1.0000000e+00, -1.6666648e-01, 8.3329687e-03, -1.9821891e-04,
          2.7130318e-06, -2.0843258e-08)
_COS_C = (1.0000000e+00, -4.9999997e-01, 4.1666560e-02, -1.3888067e-03,
          2.4774410e-05, -2.7114876e-07, 1.7351648e-09)


def _rbf_body(d2_ref, planes_ref):
    d2b = d2_ref[...]
    dist = jnp.sqrt(d2b)
    theta = np.float32(np.pi / _CUTOFF) * jnp.minimum(dist, _CUTOFF)
    t2 = theta * theta
    ps = jnp.full_like(t2, _SIN_C[-1])
    for c in _SIN_C[-2::-1]:
        ps = ps * t2 + c
    s1 = theta * ps
    pc = jnp.full_like(t2, _COS_C[-1])
    for c in _COS_C[-2::-1]:
        pc = pc * t2 + c
    c1 = pc
    env = jnp.where(dist <= _CUTOFF, 0.5 * (c1 + 1.0), 0.0)
    scale = env / dist
    sins = [s1, 2.0 * c1 * s1]
    for _ in range(3, _NRBF + 1):
        sins.append(2.0 * c1 * sins[-1] - sins[-2])
    for k in range(_NRBF):
        planes_ref[k] = sins[k] * scale
    planes_ref[_NRBF] = env


def _wsk_body(planes_ref, dw17, ws_ref):
    f32 = jnp.float32
    cn = (((0,), (0,)), ((), ()))
    for r in range(_WSR):
        S = planes_ref[:, r, :]
        ws_ref[r * 128:(r + 1) * 128, :] = lax.dot_general(
            S, dw17[...], cn, preferred_element_type=f32)


def _full(shape):
    return pl.BlockSpec(shape, lambda i: tuple(0 for _ in shape))


def _phi_call(x, w1t, b1, w2t, b2):
    return pl.pallas_call(
        _phi_body,
        grid=(_N // _BR,),
        in_specs=[
            pl.BlockSpec((_BR, _D), lambda i: (i, 0)),
            _full((_D, _D)), _full((1, _D)), _full((_D, _D)), _full((1, _D)),
        ],
        out_specs=pl.BlockSpec((_BR, _D), lambda i: (i, 0)),
        out_shape=jax.ShapeDtypeStruct((_N, _D), jnp.float32),
    )(x, w1t, b1, w2t, b2)


def _upd_call(s, acc, dw1t, db1, dw2t, db2, mw1t, mb1, mw2t, mb2):
    return pl.pallas_call(
        _upd_body,
        grid=(_N // _BR,),
        in_specs=[
            pl.BlockSpec((_BR, _D), lambda i: (i, 0)),
            pl.BlockSpec((_NC, _BR, _D), lambda i: (0, i, 0)),
            _full((_D, _D)), _full((1, _D)), _full((_D, _D)), _full((1, _D)),
            _full((_D, _D)), _full((1, _D)), _full((_D, _D)), _full((1, _D)),
        ],
        out_specs=[
            pl.BlockSpec((_BR, _D), lambda i: (i, 0)),
            pl.BlockSpec((_BR, _D), lambda i: (i, 0)),
        ],
        out_shape=[
            jax.ShapeDtypeStruct((_N, _D), jnp.float32),
            jax.ShapeDtypeStruct((_N, _D), jnp.float32),
        ],
    )(s, acc, dw1t, db1, dw2t, db2, mw1t, mb1, mw2t, mb2)


def _fin_call(vi, acc, dw1t, db1, dw2t, db2, fw1t, fb1, fw2t, fb2):
    return pl.pallas_call(
        _fin_body,
        grid=(_N // _BR,),
        in_specs=[
            pl.BlockSpec((_BR, _D), lambda i: (i, 0)),
            pl.BlockSpec((_NC, _BR, _D), lambda i: (0, i, 0)),
            _full((_D, _D)), _full((1, _D)), _full((_D, _D)), _full((1, _D)),
            _full((_D, _D)), _full((1, _D)), _full((_D, _D)), _full((1, _D)),
        ],
        out_specs=pl.BlockSpec((_BR, _D), lambda i: (i, 0)),
        out_shape=jax.ShapeDtypeStruct((_N, _D), jnp.float32),
    )(vi, acc, dw1t, db1, dw2t, db2, fw1t, fb1, fw2t, fb2)


def _rbf_call(d2):
    return pl.pallas_call(
        _rbf_body,
        grid=(_D2_ROWS // _WSR,),
        in_specs=[pl.BlockSpec((_WSR, 128), lambda i: (i, 0))],
        out_specs=pl.BlockSpec((_NRBF + 1, _WSR, 128), lambda i: (0, i, 0)),
        out_shape=jax.ShapeDtypeStruct((_NRBF + 1, _D2_ROWS, 128),
                                       jnp.float32),
    )(d2)


def _wsk_call(planes, dw17):
    return pl.pallas_call(
        _wsk_body,
        grid=(_D2_ROWS // _WSR,),
        in_specs=[
            pl.BlockSpec((_NRBF + 1, _WSR, 128), lambda i: (0, i, 0)),
            _full((_NRBF + 1, _D)),
        ],
        out_specs=pl.BlockSpec((_WSR * 128, _D), lambda i: (i, 0)),
        out_shape=jax.ShapeDtypeStruct((_EPAD, _D), jnp.float32),
    )(planes, dw17)


def kernel(cg_xyz, CG_nbr_list, mapping, S,
           im1_w1, im1_b1, im1_w2, im1_b2, im1_dw, im1_db,
           im2_w1, im2_b1, im2_w2, im2_b2, im2_dw, im2_db,
           d1_w1, d1_b1, d1_w2, d1_b2,
           d2_w1, d2_b1, d2_w2, d2_b2,
           d3_w1, d3_b1, d3_w2, d3_b2):
    f32 = jnp.float32
    x = cg_xyz[:, 0].astype(f32)
    y = cg_xyz[:, 1].astype(f32)
    z = cg_xyz[:, 2].astype(f32)
    src = CG_nbr_list[:, 0].astype(jnp.int32)
    dst = CG_nbr_list[:, 1].astype(jnp.int32)
    srcp = jnp.pad(src, (0, _EPAD - _E))
    dstp = jnp.pad(dst, (0, _EPAD - _E))

    d2 = _sc_d2()(x, y, z, srcp, dstp)

    row = lambda b: b.reshape(1, -1).astype(f32)
    planes = _rbf_call(d2)
    dw17_1 = jnp.concatenate([im1_dw.T.astype(f32), row(im1_db)], axis=0)
    dw17_2 = jnp.concatenate([im2_dw.T.astype(f32), row(im2_db)], axis=0)
    ws1 = _wsk_call(planes, dw17_1)

    phi1 = _phi_call(S, im1_w1.T, row(im1_b1), im1_w2.T, row(im1_b2))
    acc1 = _sc_edge()(phi1, ws1, src, dst)
    ws2 = _wsk_call(planes, dw17_2)

    vi, phi2 = _upd_call(S, acc1,
                         d1_w1.T, row(d1_b1), d1_w2.T, row(d1_b2),
                         im2_w1.T, row(im2_b1), im2_w2.T, row(im2_b2))
    acc2 = _sc_edge()(phi2, ws2, src, dst)

    fw1t = jnp.zeros((_D, _D), f32).at[:, :39].set(d3_w1.T)
    fb1 = jnp.zeros((1, _D), f32).at[0, :39].set(d3_b1)
    fw2t = jnp.zeros((_D, _D), f32).at[:39, :39].set(d3_w2.T)
    fb2 = jnp.zeros((1, _D), f32).at[0, :39].set(d3_b2)
    vfull = _fin_call(vi, acc2,
                      d2_w1.T, row(d2_b1), d2_w2.T, row(d2_b2),
                      fw1t, fb1, fw2t, fb2)
    V = vfull[:, :39].reshape(_N, 13, 3)
    return (None, V)
